# Initial kernel scaffold; baseline (speedup 1.0000x reference)
import jax
import jax.numpy as jnp
from jax.experimental import pallas as pl


def kernel(x, edge_index, W_self1, W_neigh1, b1, W_self2, W_neigh2, b2):
    raise NotImplementedError("write your pallas kernel here")


# speedup vs baseline: 5.1384x; 5.1384x over previous
import functools

import jax
import jax.numpy as jnp
from jax import lax
from jax.experimental import pallas as pl
from jax.experimental.pallas import tpu as pltpu
from jax.experimental.pallas import tpu_sc as plsc

N_NODES = 10000
N_EDGES = 320000
NC = 2
NS = 16
NW = NC * NS
E_PER_W = N_EDGES // NW
CHUNK = 80
N_CHUNKS = E_PER_W // CHUNK
ROWS_PER_TILE = N_NODES // NS
DUMP_CHUNK = 125
N_DUMPS = ROWS_PER_TILE // DUMP_CHUNK


def _make_sc_aggregate(d: int):
  mesh = plsc.VectorSubcoreMesh(core_axis_name="c", subcore_axis_name="s")

  @functools.partial(
      pl.kernel,
      out_type=jax.ShapeDtypeStruct((NC * N_NODES, d), jnp.float32),
      mesh=mesh,
      scratch_types=dict(
          src_v=pltpu.VMEM((CHUNK,), jnp.int32),
          dst_v=pltpu.VMEM((CHUNK,), jnp.int32),
          rows_v=pltpu.VMEM((CHUNK, d), jnp.float32),
          stage_v=pltpu.VMEM((DUMP_CHUNK, d), jnp.float32),
          acc_sh=pltpu.VMEM_SHARED((N_NODES, d), jnp.float32),
          sem=pltpu.SemaphoreType.DMA,
      ),
      compiler_params=pltpu.CompilerParams(use_tc_tiling_on_sc=False),
  )
  def agg(table_hbm, src_hbm, dst_hbm, zeros_hbm, out_hbm,
          src_v, dst_v, rows_v, stage_v, acc_sh, sem):
    c = lax.axis_index("c")
    s = lax.axis_index("s")
    wid = c * NS + s

    pltpu.sync_copy(zeros_hbm, stage_v)
    row0 = s * ROWS_PER_TILE
    for k in range(N_DUMPS):
      pltpu.sync_copy(stage_v, acc_sh.at[pl.ds(row0 + k * DUMP_CHUNK, DUMP_CHUNK)])
    plsc.subcore_barrier()

    ebase = wid * E_PER_W

    def body(i, carry):
      off = ebase + i * CHUNK
      pltpu.sync_copy(src_hbm.at[pl.ds(off, CHUNK)], src_v)
      pltpu.sync_copy(dst_hbm.at[pl.ds(off, CHUNK)], dst_v)
      pltpu.async_copy(table_hbm.at[src_v], rows_v, sem).wait()
      pltpu.sync_copy(rows_v, acc_sh.at[dst_v], add=True)
      return carry

    lax.fori_loop(0, N_CHUNKS, body, 0)
    plsc.subcore_barrier()

    out0 = c * N_NODES + row0
    for k in range(N_DUMPS):
      pltpu.sync_copy(acc_sh.at[pl.ds(row0 + k * DUMP_CHUNK, DUMP_CHUNK)], stage_v)
      pltpu.sync_copy(stage_v, out_hbm.at[pl.ds(out0 + k * DUMP_CHUNK, DUMP_CHUNK)])

  return agg


_D1 = 144
_D2 = 48
_BLK = 1000
_GRID = N_NODES // _BLK


def _dense1_body(x_ref, a0_ref, a1_ref, ws1_ref, wn1_ref, b1_ref,
                 ws2_ref, wn2_ref, b2_ref, g2p_ref, s2b_ref, dinv_ref):
  a = a0_ref[...] + a1_ref[...]
  dinv = 1.0 / jnp.maximum(a[:, 128:129], 1.0)
  mean = a[:, :128] * dinv
  h1 = x_ref[...] @ ws1_ref[...] + mean @ wn1_ref[...] + b1_ref[...]
  h1 = jnp.maximum(h1, 0.0)
  g2 = h1 @ wn2_ref[...]
  g2p_ref[...] = jnp.concatenate(
      [g2, jnp.zeros((g2.shape[0], _D2 - g2.shape[1]), jnp.float32)], axis=1)
  s2b_ref[...] = h1 @ ws2_ref[...] + b2_ref[...]
  dinv_ref[...] = jnp.broadcast_to(dinv, (dinv.shape[0], 8))


def _dense2_body(a0_ref, a1_ref, s2b_ref, dinv_ref, out_ref):
  a = a0_ref[...] + a1_ref[...]
  out_ref[...] = s2b_ref[...] + a[:, :40] * dinv_ref[:, :1]


def kernel(x, edge_index, W_self1, W_neigh1, b1, W_self2, W_neigh2, b2):
  src = edge_index[0].astype(jnp.int32)
  dst = edge_index[1].astype(jnp.int32)
  n_classes = W_self2.shape[1]

  xp = jnp.concatenate(
      [x, jnp.ones((N_NODES, 1), jnp.float32),
       jnp.zeros((N_NODES, _D1 - x.shape[1] - 1), jnp.float32)], axis=1)
  z1 = jnp.zeros((DUMP_CHUNK, _D1), jnp.float32)
  acc1 = _make_sc_aggregate(_D1)(xp, src, dst, z1)

  g2p, s2b, dinv = pl.pallas_call(
      _dense1_body,
      grid=(_GRID,),
      in_specs=[
          pl.BlockSpec((_BLK, 128), lambda i: (i, 0)),
          pl.BlockSpec((_BLK, _D1), lambda i: (i, 0)),
          pl.BlockSpec((_BLK, _D1), lambda i: (i + _GRID, 0)),
          pl.BlockSpec((128, 128), lambda i: (0, 0)),
          pl.BlockSpec((128, 128), lambda i: (0, 0)),
          pl.BlockSpec((1, 128), lambda i: (0, 0)),
          pl.BlockSpec((128, n_classes), lambda i: (0, 0)),
          pl.BlockSpec((128, n_classes), lambda i: (0, 0)),
          pl.BlockSpec((1, n_classes), lambda i: (0, 0)),
      ],
      out_specs=[
          pl.BlockSpec((_BLK, _D2), lambda i: (i, 0)),
          pl.BlockSpec((_BLK, n_classes), lambda i: (i, 0)),
          pl.BlockSpec((_BLK, 8), lambda i: (i, 0)),
      ],
      out_shape=[
          jax.ShapeDtypeStruct((N_NODES, _D2), jnp.float32),
          jax.ShapeDtypeStruct((N_NODES, n_classes), jnp.float32),
          jax.ShapeDtypeStruct((N_NODES, 8), jnp.float32),
      ],
  )(x, acc1, acc1, W_self1, W_neigh1, b1.reshape(1, -1),
    W_self2, W_neigh2, b2.reshape(1, -1))

  z2 = jnp.zeros((DUMP_CHUNK, _D2), jnp.float32)
  acc2 = _make_sc_aggregate(_D2)(g2p, src, dst, z2)

  out = pl.pallas_call(
      _dense2_body,
      grid=(_GRID,),
      in_specs=[
          pl.BlockSpec((_BLK, _D2), lambda i: (i, 0)),
          pl.BlockSpec((_BLK, _D2), lambda i: (i + _GRID, 0)),
          pl.BlockSpec((_BLK, n_classes), lambda i: (i, 0)),
          pl.BlockSpec((_BLK, 8), lambda i: (i, 0)),
      ],
      out_specs=pl.BlockSpec((_BLK, n_classes), lambda i: (i, 0)),
      out_shape=jax.ShapeDtypeStruct((N_NODES, n_classes), jnp.float32),
  )(acc2, acc2, s2b, dinv)
  return out


# speedup vs baseline: 10.8825x; 2.1179x over previous
import functools

import jax
import jax.numpy as jnp
from jax import lax
from jax.experimental import pallas as pl
from jax.experimental.pallas import tpu as pltpu
from jax.experimental.pallas import tpu_sc as plsc

N_NODES = 10000
N_EDGES = 320000
NC = 2
NS = 16
NW = NC * NS
E_PER_W = N_EDGES // NW
CHUNK = 80
N_CHUNKS = E_PER_W // CHUNK
ROWS_PER_TILE = N_NODES // NS
DUMP_CHUNK = 125
N_DUMPS = ROWS_PER_TILE // DUMP_CHUNK


def _make_sc_aggregate(d: int):
  mesh = plsc.VectorSubcoreMesh(core_axis_name="c", subcore_axis_name="s")

  @functools.partial(
      pl.kernel,
      out_type=jax.ShapeDtypeStruct((NC * N_NODES, d), jnp.float32),
      mesh=mesh,
      scratch_types=dict(
          pk_v=pltpu.VMEM((N_CHUNKS, CHUNK), jnp.int32),
          src_a=pltpu.VMEM((CHUNK,), jnp.int32),
          dst_a=pltpu.VMEM((CHUNK,), jnp.int32),
          src_b=pltpu.VMEM((CHUNK,), jnp.int32),
          dst_b=pltpu.VMEM((CHUNK,), jnp.int32),
          buf_a=pltpu.VMEM((CHUNK, d), jnp.float32),
          buf_b=pltpu.VMEM((CHUNK, d), jnp.float32),
          acc_sh=pltpu.VMEM_SHARED((N_NODES, d), jnp.float32),
          sem_a=pltpu.SemaphoreType.DMA,
          sem_b=pltpu.SemaphoreType.DMA,
      ),
      compiler_params=pltpu.CompilerParams(use_tc_tiling_on_sc=False),
  )
  def agg(table_hbm, pk_hbm, zeros_hbm, out_hbm,
          pk_v, src_a, dst_a, src_b, dst_b, buf_a, buf_b, acc_sh,
          sem_a, sem_b):
    c = lax.axis_index("c")
    s = lax.axis_index("s")
    wid = c * NS + s

    pltpu.sync_copy(pk_hbm.at[wid], pk_v)

    def unpack(i, src_x, dst_x):
      for g in range(CHUNK // 16):
        pk = pk_v[i, pl.ds(g * 16, 16)]
        src_x[pl.ds(g * 16, 16)] = lax.shift_right_logical(pk, 16)
        dst_x[pl.ds(g * 16, 16)] = lax.bitwise_and(pk, 0xFFFF)

    def gather(src_x, buf, sem):
      pltpu.async_copy(table_hbm.at[src_x], buf, sem)

    def wait(src_x, buf, sem):
      pltpu.make_async_copy(table_hbm.at[src_x], buf, sem).wait()

    def scatter(dst_x, buf):
      pltpu.sync_copy(buf, acc_sh.at[dst_x], add=True)

    unpack(0, src_a, dst_a)
    gather(src_a, buf_a, sem_a)

    pltpu.sync_copy(zeros_hbm, buf_b)
    row0 = s * ROWS_PER_TILE
    n_full, rem = divmod(ROWS_PER_TILE, CHUNK)
    for k in range(n_full):
      pltpu.sync_copy(buf_b, acc_sh.at[pl.ds(row0 + k * CHUNK, CHUNK)])
    if rem:
      pltpu.sync_copy(buf_b.at[pl.ds(0, rem)],
                      acc_sh.at[pl.ds(row0 + n_full * CHUNK, rem)])
    plsc.subcore_barrier()

    def body(j, carry):
      i0 = 2 * j
      unpack(i0 + 1, src_b, dst_b)
      gather(src_b, buf_b, sem_b)
      wait(src_a, buf_a, sem_a)
      scatter(dst_a, buf_a)
      unpack(i0 + 2, src_a, dst_a)
      gather(src_a, buf_a, sem_a)
      wait(src_b, buf_b, sem_b)
      scatter(dst_b, buf_b)
      return carry

    lax.fori_loop(0, N_CHUNKS // 2, body, 0)
    wait(src_a, buf_a, sem_a)
    scatter(dst_a, buf_a)
    plsc.subcore_barrier()

    out0 = c * N_NODES + row0
    for k in range(n_full):
      pltpu.sync_copy(acc_sh.at[pl.ds(row0 + k * CHUNK, CHUNK)], buf_a)
      pltpu.sync_copy(buf_a, out_hbm.at[pl.ds(out0 + k * CHUNK, CHUNK)])
    if rem:
      pltpu.sync_copy(acc_sh.at[pl.ds(row0 + n_full * CHUNK, rem)],
                      buf_b.at[pl.ds(0, rem)])
      pltpu.sync_copy(buf_b.at[pl.ds(0, rem)],
                      out_hbm.at[pl.ds(out0 + n_full * CHUNK, rem)])

  return agg


_D1 = 144
_D2 = 48
_BLK = 1000
_GRID = N_NODES // _BLK


def _dense1_body(x_ref, a0_ref, a1_ref, ws1_ref, wn1_ref, b1_ref,
                 ws2_ref, wn2_ref, b2_ref, g2p_ref, s2b_ref, dinv_ref):
  a = a0_ref[...] + a1_ref[...]
  dinv = 1.0 / jnp.maximum(a[:, 128:129], 1.0)
  mean = a[:, :128] * dinv
  h1 = x_ref[...] @ ws1_ref[...] + mean @ wn1_ref[...] + b1_ref[...]
  h1 = jnp.maximum(h1, 0.0)
  g2 = h1 @ wn2_ref[...]
  g2p_ref[...] = jnp.concatenate(
      [g2, jnp.zeros((g2.shape[0], _D2 - g2.shape[1]), jnp.float32)], axis=1)
  s2b_ref[...] = h1 @ ws2_ref[...] + b2_ref[...]
  dinv_ref[...] = jnp.broadcast_to(dinv, (dinv.shape[0], 8))


def _dense2_body(a0_ref, a1_ref, s2b_ref, dinv_ref, out_ref):
  a = a0_ref[...] + a1_ref[...]
  out_ref[...] = s2b_ref[...] + a[:, :40] * dinv_ref[:, :1]


def kernel(x, edge_index, W_self1, W_neigh1, b1, W_self2, W_neigh2, b2):
  src = edge_index[0].astype(jnp.int32)
  dst = edge_index[1].astype(jnp.int32)
  pk = ((src << 16) | dst).reshape(NW, N_CHUNKS, CHUNK)
  n_classes = W_self2.shape[1]

  xp = jnp.concatenate(
      [x, jnp.ones((N_NODES, 1), jnp.float32),
       jnp.zeros((N_NODES, _D1 - x.shape[1] - 1), jnp.float32)], axis=1)
  z1 = jnp.zeros((CHUNK, _D1), jnp.float32)
  acc1 = _make_sc_aggregate(_D1)(xp, pk, z1)

  g2p, s2b, dinv = pl.pallas_call(
      _dense1_body,
      grid=(_GRID,),
      in_specs=[
          pl.BlockSpec((_BLK, 128), lambda i: (i, 0)),
          pl.BlockSpec((_BLK, _D1), lambda i: (i, 0)),
          pl.BlockSpec((_BLK, _D1), lambda i: (i + _GRID, 0)),
          pl.BlockSpec((128, 128), lambda i: (0, 0)),
          pl.BlockSpec((128, 128), lambda i: (0, 0)),
          pl.BlockSpec((1, 128), lambda i: (0, 0)),
          pl.BlockSpec((128, n_classes), lambda i: (0, 0)),
          pl.BlockSpec((128, n_classes), lambda i: (0, 0)),
          pl.BlockSpec((1, n_classes), lambda i: (0, 0)),
      ],
      out_specs=[
          pl.BlockSpec((_BLK, _D2), lambda i: (i, 0)),
          pl.BlockSpec((_BLK, n_classes), lambda i: (i, 0)),
          pl.BlockSpec((_BLK, 8), lambda i: (i, 0)),
      ],
      out_shape=[
          jax.ShapeDtypeStruct((N_NODES, _D2), jnp.float32),
          jax.ShapeDtypeStruct((N_NODES, n_classes), jnp.float32),
          jax.ShapeDtypeStruct((N_NODES, 8), jnp.float32),
      ],
  )(x, acc1, acc1, W_self1, W_neigh1, b1.reshape(1, -1),
    W_self2, W_neigh2, b2.reshape(1, -1))

  z2 = jnp.zeros((CHUNK, _D2), jnp.float32)
  acc2 = _make_sc_aggregate(_D2)(g2p, pk, z2)

  out = pl.pallas_call(
      _dense2_body,
      grid=(_GRID,),
      in_specs=[
          pl.BlockSpec((_BLK, _D2), lambda i: (i, 0)),
          pl.BlockSpec((_BLK, _D2), lambda i: (i + _GRID, 0)),
          pl.BlockSpec((_BLK, n_classes), lambda i: (i, 0)),
          pl.BlockSpec((_BLK, 8), lambda i: (i, 0)),
      ],
      out_specs=pl.BlockSpec((_BLK, n_classes), lambda i: (i, 0)),
      out_shape=jax.ShapeDtypeStruct((N_NODES, n_classes), jnp.float32),
  )(acc2, acc2, s2b, dinv)
  return out
